# baseline (speedup 1.0000x reference)
import functools

import jax
import jax.numpy as jnp
from jax import lax
from jax.experimental import pallas as pl
from jax.experimental.pallas import tpu as pltpu
from jax.experimental.pallas import tpu_sc as plsc

N = 10000
E = 320000
D = 128
DE = 16
H = 256
HH = 128
L = 5
G = 256
OUT = 256
LN_EPS = 1e-5

NC = 2
NS = 16
CHUNK = 80
NBUF = 3
SC_ITERS = 252
TRIPLES = SC_ITERS // NBUF
EPAD = SC_ITERS * NS * CHUNK
NPAD = N + 16
STRIPE = 624
TAIL0 = NS * STRIPE
TAILN = N - TAIL0

f32 = jnp.float32


def _prep_body(we_r, wle_r, be_r, ble_r, wf_r, bf_r):
    wf_r[0] = jnp.dot(we_r[...], wle_r[0], preferred_element_type=f32)
    bf_r[0] = (jnp.dot(be_r[...], wle_r[0], preferred_element_type=f32)
               + ble_r[0])


def _fuse_weights(We, be2, Wle, ble3):
    return pl.pallas_call(
        _prep_body,
        grid=(L,),
        in_specs=[
            pl.BlockSpec((DE, H), lambda l: (0, 0)),
            pl.BlockSpec((1, H, H), lambda l: (l, 0, 0)),
            pl.BlockSpec((1, H), lambda l: (0, 0)),
            pl.BlockSpec((1, 1, H), lambda l: (l, 0, 0)),
        ],
        out_specs=[
            pl.BlockSpec((1, DE, H), lambda l: (l, 0, 0)),
            pl.BlockSpec((1, 1, H), lambda l: (l, 0, 0)),
        ],
        out_shape=[
            jax.ShapeDtypeStruct((L, DE, H), f32),
            jax.ShapeDtypeStruct((L, 1, H), f32),
        ],
    )(We, Wle, be2, ble3)


_BN = 1000
_NBLK = N // _BN


def _hinit_body(x_r, wn_r, bn_r, o_r):
    o_r[...] = (jnp.dot(x_r[...], wn_r[...], preferred_element_type=f32)
                + bn_r[...])


def _h_init(x, Wn, bn2):
    return pl.pallas_call(
        _hinit_body,
        grid=(_NBLK,),
        in_specs=[
            pl.BlockSpec((_BN, D), lambda i: (i, 0)),
            pl.BlockSpec((D, H), lambda i: (0, 0)),
            pl.BlockSpec((1, H), lambda i: (0, 0)),
        ],
        out_specs=pl.BlockSpec((_BN, H), lambda i: (i, 0)),
        out_shape=jax.ShapeDtypeStruct((N, H), f32),
    )(x, Wn, bn2)


_BE = 2000
_GE = E // _BE


def _ep_body(ea_r, wf_r, bf_r, o_r):
    o_r[0, 0] = (jnp.dot(ea_r[...], wf_r[0], preferred_element_type=f32)
                 + bf_r[0, 0]).astype(jnp.bfloat16)


def _edge_proj(edge_attr, Wf, bf):
    return pl.pallas_call(
        _ep_body,
        grid=(_GE, L, NC),
        in_specs=[
            pl.BlockSpec((_BE, DE), lambda i, l, c: (i, 0)),
            pl.BlockSpec((1, DE, HH), lambda i, l, c: (l, 0, c)),
            pl.BlockSpec((1, 1, 1, HH), lambda i, l, c: (l, c, 0, 0)),
        ],
        out_specs=pl.BlockSpec((1, 1, _BE, HH), lambda i, l, c: (l, c, i, 0)),
        out_shape=jax.ShapeDtypeStruct((L, NC, E, HH), jnp.bfloat16),
    )(edge_attr, Wf, bf.reshape(L, NC, 1, HH))


def _make_sc_layer(layer):
    mesh = plsc.VectorSubcoreMesh(core_axis_name="c", subcore_axis_name="s",
                                  num_cores=NC, num_subcores=NS)

    @functools.partial(
        pl.kernel,
        out_type=jax.ShapeDtypeStruct((NC * N, HH), f32),
        mesh=mesh,
        scratch_types=[
            [pltpu.VMEM((2, CHUNK), jnp.int32) for _ in range(NBUF)],
            [pltpu.VMEM((CHUNK,), jnp.int32) for _ in range(NBUF)],
            [pltpu.VMEM((CHUNK,), jnp.int32) for _ in range(NBUF)],
            [pltpu.VMEM((CHUNK, HH), f32) for _ in range(NBUF)],
            [pltpu.VMEM((CHUNK // 2, 2, HH), jnp.bfloat16)
             for _ in range(NBUF)],
            pltpu.VMEM_SHARED((NPAD, HH), f32),
            [pltpu.SemaphoreType.DMA for _ in range(NBUF)],
            [pltpu.SemaphoreType.DMA for _ in range(NBUF)],
            [pltpu.SemaphoreType.DMA for _ in range(NBUF)],
            [pltpu.SemaphoreType.DMA for _ in range(NBUF)],
        ],
    )
    def sc_layer(h2_hbm, ep_hbm, idx_hbm, out_hbm,
                 ibuf, gidx, didx, gbuf, ebuf, aggs, isem, gsem, esem, ssem):
        c = lax.axis_index("c")
        s = lax.axis_index("s")
        ep_base2 = layer * E + c * (E // 2)

        def stage_a(j, p):
            pltpu.async_copy(idx_hbm.at[j * NS + s], ibuf[p], isem[p])

        def fetch(j, p, drain):
            if drain:
                pltpu.make_async_copy(gbuf[p], aggs.at[didx[p]],
                                      ssem[p]).wait()
            pltpu.make_async_copy(idx_hbm.at[j * NS + s], ibuf[p],
                                  isem[p]).wait()
            for k in range(CHUNK // 16):
                gidx[p][pl.ds(k * 16, 16)] = ibuf[p][0, pl.ds(k * 16, 16)] + c
                didx[p][pl.ds(k * 16, 16)] = ibuf[p][1, pl.ds(k * 16, 16)]
            pltpu.async_copy(h2_hbm.at[gidx[p]], gbuf[p], gsem[p])
            epb2 = jnp.minimum((j * NS + s) * (CHUNK // 2),
                               (E - CHUNK) // 2)
            pltpu.async_copy(ep_hbm.at[pl.ds(ep_base2 + epb2, CHUNK // 2)],
                             ebuf[p], esem[p])

            @pl.when(j + NBUF < SC_ITERS)
            def _():
                stage_a(j + NBUF, p)

        def process(j, p):
            epb2 = jnp.minimum((j * NS + s) * (CHUNK // 2), (E - CHUNK) // 2)
            pltpu.make_async_copy(h2_hbm.at[gidx[p]], gbuf[p], gsem[p]).wait()
            pltpu.make_async_copy(ep_hbm.at[pl.ds(ep_base2 + epb2, CHUNK // 2)],
                                  ebuf[p], esem[p]).wait()

            def rbody(q, rc):
                for rr in range(2):
                    e = ebuf[p][q, rr, :].astype(f32)
                    g = gbuf[p][2 * q + rr, :]
                    gbuf[p][2 * q + rr, :] = jnp.maximum(g + e, 0.0)
                return rc
            lax.fori_loop(0, CHUNK // 2, rbody, 0)
            pltpu.async_copy(gbuf[p], aggs.at[didx[p]], ssem[p], add=True)

        zbuf = gbuf[0]

        def zrow(r, carry):
            for k in range(HH // 16):
                zbuf[r, pl.ds(k * 16, 16)] = jnp.zeros((16,), f32)
            return carry
        lax.fori_loop(0, CHUNK, zrow, 0)
        row0 = s * STRIPE

        def zero_span(start, nrows):
            full, rem = divmod(nrows, CHUNK)
            for k in range(full):
                pltpu.sync_copy(zbuf, aggs.at[pl.ds(start + k * CHUNK, CHUNK)])
            if rem:
                pltpu.sync_copy(zbuf.at[pl.ds(0, rem)],
                                aggs.at[pl.ds(start + full * CHUNK, rem)])

        zero_span(row0, STRIPE)

        @pl.when(s == NS - 1)
        def _():
            zero_span(TAIL0, TAILN + NPAD - N)

        for p in range(NBUF):
            stage_a(p, p)
        for p in range(NBUF):
            fetch(p, p, drain=False)
        plsc.subcore_barrier()

        def triple_body(ip, carry):
            for p in range(NBUF):
                process(ip * NBUF + p, p)

            @pl.when(ip < TRIPLES - 1)
            def _():
                for p in range(NBUF):
                    fetch(ip * NBUF + NBUF + p, p, drain=True)
            return carry

        lax.fori_loop(0, TRIPLES, triple_body, 0)
        for p in range(NBUF):
            pltpu.make_async_copy(gbuf[p], aggs.at[didx[p]], ssem[p]).wait()
        plsc.subcore_barrier()
        pltpu.sync_copy(aggs.at[pl.ds(row0, STRIPE)],
                        out_hbm.at[pl.ds(c * N + row0, STRIPE)])

        @pl.when(s == NS - 1)
        def _():
            pltpu.sync_copy(aggs.at[pl.ds(TAIL0, TAILN)],
                            out_hbm.at[pl.ds(c * N + TAIL0, TAILN)])

    return sc_layer


_SC_LAYERS = [_make_sc_layer(l) for l in range(L)]


def _mlp_body(h_r, a0_r, a1_r, w1_r, b1_r, w2_r, b2_r, g_r, be_r, o_r):
    z = jnp.concatenate(
        [h_r[:, :HH] + a0_r[...], h_r[:, HH:] + a1_r[...]], axis=1)
    t = jnp.maximum(jnp.dot(z, w1_r[0], preferred_element_type=f32)
                    + b1_r[0], 0.0)
    u = jnp.dot(t, w2_r[0], preferred_element_type=f32) + b2_r[0]
    mu = jnp.mean(u, axis=1, keepdims=True)
    var = jnp.mean((u - mu) ** 2, axis=1, keepdims=True)
    ln = (u - mu) * lax.rsqrt(var + LN_EPS) * g_r[0] + be_r[0]
    o_r[...] = jnp.maximum(ln, 0.0)


def _node_mlp(layer, h, agg2, W1, b13, W2, b23, gamma3, beta3):
    vec_spec = pl.BlockSpec((1, 1, H), lambda i, l=layer: (l, 0, 0))
    mat_spec = pl.BlockSpec((1, H, H), lambda i, l=layer: (l, 0, 0))
    return pl.pallas_call(
        _mlp_body,
        grid=(_NBLK,),
        in_specs=[
            pl.BlockSpec((_BN, H), lambda i: (i, 0)),
            pl.BlockSpec((_BN, HH), lambda i: (i, 0)),
            pl.BlockSpec((_BN, HH), lambda i: (i + _NBLK, 0)),
            mat_spec, vec_spec, mat_spec, vec_spec, vec_spec, vec_spec,
        ],
        out_specs=pl.BlockSpec((_BN, H), lambda i: (i, 0)),
        out_shape=jax.ShapeDtypeStruct((N, H), f32),
    )(h, agg2, agg2, W1, b13, W2, b23, gamma3, beta3)


def _pool_body(h_r, b_r, wr1_r, br1_r, wr2_r, br2_r, o_r, acc):
    i = pl.program_id(0)

    @pl.when(i == 0)
    def _():
        acc[...] = jnp.zeros((G, H), f32)

    onehot = (b_r[...] == lax.broadcasted_iota(jnp.int32, (_BN, G), 1)
              ).astype(f32)
    acc[...] += lax.dot_general(onehot, h_r[...],
                                (((0,), (0,)), ((), ())),
                                preferred_element_type=f32)

    @pl.when(i == _NBLK - 1)
    def _():
        g = acc[...]
        t = jnp.maximum(jnp.dot(g, wr1_r[...], preferred_element_type=f32)
                        + br1_r[...], 0.0)
        o_r[...] = (jnp.dot(t, wr2_r[...], preferred_element_type=f32)
                    + br2_r[...])


def _pool_readout(h, batch_col, Wr1, br1_2, Wr2, br2_2):
    return pl.pallas_call(
        _pool_body,
        grid=(_NBLK,),
        in_specs=[
            pl.BlockSpec((_BN, H), lambda i: (i, 0)),
            pl.BlockSpec((_BN, 1), lambda i: (i, 0)),
            pl.BlockSpec((H, H), lambda i: (0, 0)),
            pl.BlockSpec((1, H), lambda i: (0, 0)),
            pl.BlockSpec((H, OUT), lambda i: (0, 0)),
            pl.BlockSpec((1, OUT), lambda i: (0, 0)),
        ],
        out_specs=pl.BlockSpec((G, OUT), lambda i: (0, 0)),
        out_shape=jax.ShapeDtypeStruct((G, OUT), f32),
        scratch_shapes=[pltpu.VMEM((G, H), f32)],
    )(h, batch_col, Wr1, br1_2, Wr2, br2_2)


def kernel(x, edge_index, edge_attr, batch, Wn, bn, We, be, Wle, ble,
           W1, b1, W2, b2, gamma, beta, Wr1, br1, Wr2, br2):
    src = jnp.pad(edge_index[0], (0, EPAD - E))
    dst = jnp.pad(edge_index[1], (0, EPAD - E), constant_values=N)
    packed_idx = jnp.stack([(2 * src).reshape(EPAD // CHUNK, CHUNK),
                            dst.reshape(EPAD // CHUNK, CHUNK)], axis=1)

    Wf, bf = _fuse_weights(We, be.reshape(1, H), Wle, ble.reshape(L, 1, H))
    ep = _edge_proj(edge_attr, Wf, bf)
    ep_flat = ep.reshape(L * NC * E // 2, 2, HH)

    b13 = b1.reshape(L, 1, H)
    b23 = b2.reshape(L, 1, H)
    gamma3 = gamma.reshape(L, 1, H)
    beta3 = beta.reshape(L, 1, H)

    h = _h_init(x, Wn, bn.reshape(1, H))
    for l in range(L):
        h2 = h.reshape(NC * N, HH)
        agg2 = _SC_LAYERS[l](h2, ep_flat, packed_idx)
        h = _node_mlp(l, h, agg2, W1, b13, W2, b23, gamma3, beta3)

    return _pool_readout(h, batch.reshape(N, 1), Wr1,
                         br1.reshape(1, H), Wr2, br2.reshape(1, OUT))


# speedup vs baseline: 1.2146x; 1.2146x over previous
import functools

import jax
import jax.numpy as jnp
from jax import lax
from jax.experimental import pallas as pl
from jax.experimental.pallas import tpu as pltpu
from jax.experimental.pallas import tpu_sc as plsc

N = 10000
E = 320000
D = 128
DE = 16
H = 256
HH = 128
L = 5
G = 256
OUT = 256
LN_EPS = 1e-5

NC = 2
NS = 16
CHUNK = 80
NBUF = 3
SC_ITERS = 252
TRIPLES = SC_ITERS // NBUF
EPAD = SC_ITERS * NS * CHUNK
NPAD = N + 16
STRIPE = 624
TAIL0 = NS * STRIPE
TAILN = N - TAIL0

f32 = jnp.float32


def _prep_body(we_r, wle_r, be_r, ble_r, wf_r, bf_r):
    wf_r[0] = jnp.dot(we_r[...], wle_r[0], preferred_element_type=f32)
    bf_r[0] = (jnp.dot(be_r[...], wle_r[0], preferred_element_type=f32)
               + ble_r[0])


def _fuse_weights(We, be2, Wle, ble3):
    return pl.pallas_call(
        _prep_body,
        grid=(L,),
        in_specs=[
            pl.BlockSpec((DE, H), lambda l: (0, 0)),
            pl.BlockSpec((1, H, H), lambda l: (l, 0, 0)),
            pl.BlockSpec((1, H), lambda l: (0, 0)),
            pl.BlockSpec((1, 1, H), lambda l: (l, 0, 0)),
        ],
        out_specs=[
            pl.BlockSpec((1, DE, H), lambda l: (l, 0, 0)),
            pl.BlockSpec((1, 1, H), lambda l: (l, 0, 0)),
        ],
        out_shape=[
            jax.ShapeDtypeStruct((L, DE, H), f32),
            jax.ShapeDtypeStruct((L, 1, H), f32),
        ],
    )(We, Wle, be2, ble3)


_BN = 1000
_NBLK = N // _BN


def _hinit_body(x_r, wn_r, bn_r, o_r):
    o_r[...] = (jnp.dot(x_r[...], wn_r[...], preferred_element_type=f32)
                + bn_r[...])


def _h_init(x, Wn, bn2):
    return pl.pallas_call(
        _hinit_body,
        grid=(_NBLK,),
        in_specs=[
            pl.BlockSpec((_BN, D), lambda i: (i, 0)),
            pl.BlockSpec((D, H), lambda i: (0, 0)),
            pl.BlockSpec((1, H), lambda i: (0, 0)),
        ],
        out_specs=pl.BlockSpec((_BN, H), lambda i: (i, 0)),
        out_shape=jax.ShapeDtypeStruct((N, H), f32),
    )(x, Wn, bn2)


_BE = 2000
_GE = E // _BE


def _ep_body(ea_r, wf_r, bf_r, o_r):
    o_r[0] = (jnp.dot(ea_r[...], wf_r[0], preferred_element_type=f32)
              + bf_r[0, 0]).astype(jnp.bfloat16)


def _edge_proj(edge_attr, Wf, bf, layer):
    return pl.pallas_call(
        _ep_body,
        grid=(_GE, NC),
        in_specs=[
            pl.BlockSpec((_BE, DE), lambda i, c: (i, 0)),
            pl.BlockSpec((1, DE, HH), lambda i, c, l=layer: (l, 0, c)),
            pl.BlockSpec((1, 1, 1, HH), lambda i, c, l=layer: (l, c, 0, 0)),
        ],
        out_specs=pl.BlockSpec((1, _BE, HH), lambda i, c: (c, i, 0)),
        out_shape=jax.ShapeDtypeStruct((NC, E, HH), jnp.bfloat16),
    )(edge_attr, Wf, bf.reshape(L, NC, 1, HH))


def _make_sc_layer():
    mesh = plsc.VectorSubcoreMesh(core_axis_name="c", subcore_axis_name="s",
                                  num_cores=NC, num_subcores=NS)

    @functools.partial(
        pl.kernel,
        out_type=jax.ShapeDtypeStruct((NC * N, HH), f32),
        mesh=mesh,
        scratch_types=[
            [pltpu.VMEM((2, CHUNK), jnp.int32) for _ in range(NBUF)],
            [pltpu.VMEM((CHUNK,), jnp.int32) for _ in range(NBUF)],
            [pltpu.VMEM((CHUNK,), jnp.int32) for _ in range(NBUF)],
            [pltpu.VMEM((CHUNK, HH), f32) for _ in range(NBUF)],
            [pltpu.VMEM((CHUNK // 2, 2, HH), jnp.bfloat16)
             for _ in range(NBUF)],
            pltpu.VMEM_SHARED((NPAD, HH), f32),
            [pltpu.SemaphoreType.DMA for _ in range(NBUF)],
            [pltpu.SemaphoreType.DMA for _ in range(NBUF)],
            [pltpu.SemaphoreType.DMA for _ in range(NBUF)],
            [pltpu.SemaphoreType.DMA for _ in range(NBUF)],
        ],
    )
    def sc_layer(h2_hbm, ep_hbm, idx_hbm, out_hbm,
                 ibuf, gidx, didx, gbuf, ebuf, aggs, isem, gsem, esem, ssem):
        c = lax.axis_index("c")
        s = lax.axis_index("s")
        ep_base2 = c * (E // 2)

        def stage_a(j, p):
            pltpu.async_copy(idx_hbm.at[j * NS + s], ibuf[p], isem[p])

        def fetch(j, p, drain):
            if drain:
                pltpu.make_async_copy(gbuf[p], aggs.at[didx[p]],
                                      ssem[p]).wait()
            pltpu.make_async_copy(idx_hbm.at[j * NS + s], ibuf[p],
                                  isem[p]).wait()
            for k in range(CHUNK // 16):
                gidx[p][pl.ds(k * 16, 16)] = ibuf[p][0, pl.ds(k * 16, 16)] + c
                didx[p][pl.ds(k * 16, 16)] = ibuf[p][1, pl.ds(k * 16, 16)]
            pltpu.async_copy(h2_hbm.at[gidx[p]], gbuf[p], gsem[p])
            epb2 = jnp.minimum((j * NS + s) * (CHUNK // 2),
                               (E - CHUNK) // 2)
            pltpu.async_copy(ep_hbm.at[pl.ds(ep_base2 + epb2, CHUNK // 2)],
                             ebuf[p], esem[p])

            @pl.when(j + NBUF < SC_ITERS)
            def _():
                stage_a(j + NBUF, p)

        def process(j, p):
            epb2 = jnp.minimum((j * NS + s) * (CHUNK // 2), (E - CHUNK) // 2)
            pltpu.make_async_copy(h2_hbm.at[gidx[p]], gbuf[p], gsem[p]).wait()
            pltpu.make_async_copy(ep_hbm.at[pl.ds(ep_base2 + epb2, CHUNK // 2)],
                                  ebuf[p], esem[p]).wait()

            def rbody(q, rc):
                for rr in range(2):
                    e = ebuf[p][q, rr, :].astype(f32)
                    g = gbuf[p][2 * q + rr, :]
                    gbuf[p][2 * q + rr, :] = jnp.maximum(g + e, 0.0)
                return rc
            lax.fori_loop(0, CHUNK // 2, rbody, 0)
            pltpu.async_copy(gbuf[p], aggs.at[didx[p]], ssem[p], add=True)

        zbuf = gbuf[0]

        def zrow(r, carry):
            for k in range(HH // 16):
                zbuf[r, pl.ds(k * 16, 16)] = jnp.zeros((16,), f32)
            return carry
        lax.fori_loop(0, CHUNK, zrow, 0)
        row0 = s * STRIPE

        def zero_span(start, nrows):
            full, rem = divmod(nrows, CHUNK)
            for k in range(full):
                pltpu.sync_copy(zbuf, aggs.at[pl.ds(start + k * CHUNK, CHUNK)])
            if rem:
                pltpu.sync_copy(zbuf.at[pl.ds(0, rem)],
                                aggs.at[pl.ds(start + full * CHUNK, rem)])

        zero_span(row0, STRIPE)

        @pl.when(s == NS - 1)
        def _():
            zero_span(TAIL0, TAILN + NPAD - N)

        for p in range(NBUF):
            stage_a(p, p)
        for p in range(NBUF):
            fetch(p, p, drain=False)
        plsc.subcore_barrier()

        def triple_body(ip, carry):
            for p in range(NBUF):
                process(ip * NBUF + p, p)

            @pl.when(ip < TRIPLES - 1)
            def _():
                for p in range(NBUF):
                    fetch(ip * NBUF + NBUF + p, p, drain=True)
            return carry

        lax.fori_loop(0, TRIPLES, triple_body, 0)
        for p in range(NBUF):
            pltpu.make_async_copy(gbuf[p], aggs.at[didx[p]], ssem[p]).wait()
        plsc.subcore_barrier()
        pltpu.sync_copy(aggs.at[pl.ds(row0, STRIPE)],
                        out_hbm.at[pl.ds(c * N + row0, STRIPE)])

        @pl.when(s == NS - 1)
        def _():
            pltpu.sync_copy(aggs.at[pl.ds(TAIL0, TAILN)],
                            out_hbm.at[pl.ds(c * N + TAIL0, TAILN)])

    return sc_layer


_SC_LAYER = _make_sc_layer()


def _mlp_body(h_r, a0_r, a1_r, w1_r, b1_r, w2_r, b2_r, g_r, be_r, o_r):
    z = jnp.concatenate(
        [h_r[:, :HH] + a0_r[...], h_r[:, HH:] + a1_r[...]], axis=1)
    t = jnp.maximum(jnp.dot(z, w1_r[0], preferred_element_type=f32)
                    + b1_r[0], 0.0)
    u = jnp.dot(t, w2_r[0], preferred_element_type=f32) + b2_r[0]
    mu = jnp.mean(u, axis=1, keepdims=True)
    var = jnp.mean((u - mu) ** 2, axis=1, keepdims=True)
    ln = (u - mu) * lax.rsqrt(var + LN_EPS) * g_r[0] + be_r[0]
    o_r[...] = jnp.maximum(ln, 0.0)


def _node_mlp(layer, h, agg2, W1, b13, W2, b23, gamma3, beta3):
    vec_spec = pl.BlockSpec((1, 1, H), lambda i, l=layer: (l, 0, 0))
    mat_spec = pl.BlockSpec((1, H, H), lambda i, l=layer: (l, 0, 0))
    return pl.pallas_call(
        _mlp_body,
        grid=(_NBLK,),
        in_specs=[
            pl.BlockSpec((_BN, H), lambda i: (i, 0)),
            pl.BlockSpec((_BN, HH), lambda i: (i, 0)),
            pl.BlockSpec((_BN, HH), lambda i: (i + _NBLK, 0)),
            mat_spec, vec_spec, mat_spec, vec_spec, vec_spec, vec_spec,
        ],
        out_specs=pl.BlockSpec((_BN, H), lambda i: (i, 0)),
        out_shape=jax.ShapeDtypeStruct((N, H), f32),
    )(h, agg2, agg2, W1, b13, W2, b23, gamma3, beta3)


def _pool_body(h_r, b_r, wr1_r, br1_r, wr2_r, br2_r, o_r, acc):
    i = pl.program_id(0)

    @pl.when(i == 0)
    def _():
        acc[...] = jnp.zeros((G, H), f32)

    onehot = (b_r[...] == lax.broadcasted_iota(jnp.int32, (_BN, G), 1)
              ).astype(f32)
    acc[...] += lax.dot_general(onehot, h_r[...],
                                (((0,), (0,)), ((), ())),
                                preferred_element_type=f32)

    @pl.when(i == _NBLK - 1)
    def _():
        g = acc[...]
        t = jnp.maximum(jnp.dot(g, wr1_r[...], preferred_element_type=f32)
                        + br1_r[...], 0.0)
        o_r[...] = (jnp.dot(t, wr2_r[...], preferred_element_type=f32)
                    + br2_r[...])


def _pool_readout(h, batch_col, Wr1, br1_2, Wr2, br2_2):
    return pl.pallas_call(
        _pool_body,
        grid=(_NBLK,),
        in_specs=[
            pl.BlockSpec((_BN, H), lambda i: (i, 0)),
            pl.BlockSpec((_BN, 1), lambda i: (i, 0)),
            pl.BlockSpec((H, H), lambda i: (0, 0)),
            pl.BlockSpec((1, H), lambda i: (0, 0)),
            pl.BlockSpec((H, OUT), lambda i: (0, 0)),
            pl.BlockSpec((1, OUT), lambda i: (0, 0)),
        ],
        out_specs=pl.BlockSpec((G, OUT), lambda i: (0, 0)),
        out_shape=jax.ShapeDtypeStruct((G, OUT), f32),
        scratch_shapes=[pltpu.VMEM((G, H), f32)],
    )(h, batch_col, Wr1, br1_2, Wr2, br2_2)


def kernel(x, edge_index, edge_attr, batch, Wn, bn, We, be, Wle, ble,
           W1, b1, W2, b2, gamma, beta, Wr1, br1, Wr2, br2):
    src = jnp.pad(edge_index[0], (0, EPAD - E))
    dst = jnp.pad(edge_index[1], (0, EPAD - E), constant_values=N)
    packed_idx = jnp.stack([(2 * src).reshape(EPAD // CHUNK, CHUNK),
                            dst.reshape(EPAD // CHUNK, CHUNK)], axis=1)

    Wf, bf = _fuse_weights(We, be.reshape(1, H), Wle, ble.reshape(L, 1, H))

    b13 = b1.reshape(L, 1, H)
    b23 = b2.reshape(L, 1, H)
    gamma3 = gamma.reshape(L, 1, H)
    beta3 = beta.reshape(L, 1, H)

    h = _h_init(x, Wn, bn.reshape(1, H))
    ep = _edge_proj(edge_attr, Wf, bf, 0)
    for l in range(L):
        h2 = h.reshape(NC * N, HH)
        agg2 = _SC_LAYER(h2, ep.reshape(NC * E // 2, 2, HH), packed_idx)
        if l + 1 < L:
            ep = _edge_proj(edge_attr, Wf, bf, l + 1)
        h = _node_mlp(l, h, agg2, W1, b13, W2, b23, gamma3, beta3)

    return _pool_readout(h, batch.reshape(N, 1), Wr1,
                         br1.reshape(1, H), Wr2, br2.reshape(1, OUT))
